# baseline (speedup 1.0000x reference)
import functools

import jax
import jax.numpy as jnp
from jax import lax
from jax.experimental import pallas as pl
from jax.experimental.pallas import tpu as pltpu
from jax.experimental.pallas import tpu_sc as plsc

B = 16384
F = 26
DENSE = 13
V = 100000
H = 32
EMB_W = F * H
LOOKUPS = B * F

NC = 2
NS = 16
NW = NC * NS
CHUNK = 128
CPW = B // CHUNK // NW

BM = 1024
NB = B // BM


ROWS_W = B // NW
NL = 16


def _sc_gather(tables, idxp):
    mesh = plsc.VectorSubcoreMesh(core_axis_name="c", subcore_axis_name="s")

    @functools.partial(
        pl.kernel,
        mesh=mesh,
        compiler_params=pltpu.CompilerParams(
            use_tc_tiling_on_sc=False, needs_layout_passes=False),
        out_type=jax.ShapeDtypeStruct((B, EMB_W), jnp.float32),
        scratch_types=[
            pltpu.VMEM((ROWS_W, 128), jnp.int32),
            pltpu.VMEM((F, CPW, CHUNK), jnp.int32),
            pltpu.VMEM((CHUNK, H), jnp.float32),
            pltpu.SemaphoreType.DMA,
        ],
    )
    def k(tab_hbm, x_hbm, out_hbm, xv, idx_v, rows, sem):
        wid = lax.axis_index("s") * NC + lax.axis_index("c")
        b0 = wid * ROWS_W
        pltpu.sync_copy(x_hbm.at[pl.ds(b0, ROWS_W)], xv)

        lane = lax.iota(jnp.int32, NL)

        def build(t, carry):
            f = t // (ROWS_W // NL)
            g = t % (ROWS_W // NL)
            ids = plsc.load_gather(
                xv, [g * NL + lane, jnp.full((NL,), f, jnp.int32)])
            idx_v[f, g // (CHUNK // NL),
                  pl.ds((g % (CHUNK // NL)) * NL, NL)] = ids
            return carry

        lax.fori_loop(0, F * (ROWS_W // NL), build, 0)

        def task(t, carry):
            f = t // CPW
            j = t % CPW
            pltpu.async_copy(tab_hbm.at[f].at[idx_v.at[f, j]], rows, sem).wait()
            pltpu.sync_copy(
                rows,
                out_hbm.at[pl.ds(b0 + j * CHUNK, CHUNK), pl.ds(f * H, H)])
            return carry

        lax.fori_loop(0, F * CPW, task, 0)

    return k(tables, idxp)


def _mlp1_body(emb_ref, xd_ref, w1e_ref, w1d_ref, z_ref, stats_ref):
    j = pl.program_id(0)
    z = jnp.dot(emb_ref[...], w1e_ref[...], preferred_element_type=jnp.float32)
    z = z + jnp.dot(xd_ref[...], w1d_ref[...], preferred_element_type=jnp.float32)
    z_ref[...] = z
    s1 = jnp.sum(z, axis=0, keepdims=True)
    s2 = jnp.sum(z * z, axis=0, keepdims=True)

    @pl.when(j == 0)
    def _():
        stats_ref[...] = jnp.zeros_like(stats_ref)

    stats_ref[...] += jnp.concatenate(
        [s1, s2, jnp.zeros((6, 128), jnp.float32)], axis=0)


def _mlp1(emb, xdp, w1e, w1dp):
    return pl.pallas_call(
        _mlp1_body,
        grid=(NB,),
        in_specs=[
            pl.BlockSpec((BM, EMB_W), lambda j: (j, 0)),
            pl.BlockSpec((BM, 16), lambda j: (j, 0)),
            pl.BlockSpec((EMB_W, 128), lambda j: (0, 0)),
            pl.BlockSpec((16, 128), lambda j: (0, 0)),
        ],
        out_specs=[
            pl.BlockSpec((BM, 128), lambda j: (j, 0)),
            pl.BlockSpec((8, 128), lambda j: (0, 0)),
        ],
        out_shape=[
            jax.ShapeDtypeStruct((B, 128), jnp.float32),
            jax.ShapeDtypeStruct((8, 128), jnp.float32),
        ],
    )(emb, xdp, w1e, w1dp)


def _mlp2_body(z_ref, stats_ref, gb_ref, w2_ref, b2_ref, out_ref):
    stats = stats_ref[...]
    mean = stats[0:1] / B
    var = stats[1:2] / B - mean * mean
    scale = gb_ref[0:1] * lax.rsqrt(var + 1e-5)
    shift = gb_ref[1:2] - mean * scale
    a = jnp.maximum(z_ref[...] * scale + shift, 0.0)
    o = jnp.sum(a * w2_ref[...], axis=1, keepdims=True) + b2_ref[...]
    out_ref[...] = jax.nn.sigmoid(o)


def _mlp2(z, stats, gb, w2row, b2):
    return pl.pallas_call(
        _mlp2_body,
        grid=(NB,),
        in_specs=[
            pl.BlockSpec((BM, 128), lambda j: (j, 0)),
            pl.BlockSpec((8, 128), lambda j: (0, 0)),
            pl.BlockSpec((2, 128), lambda j: (0, 0)),
            pl.BlockSpec((1, 128), lambda j: (0, 0)),
            pl.BlockSpec((1, 1), lambda j: (0, 0)),
        ],
        out_specs=pl.BlockSpec((BM, 1), lambda j: (j, 0)),
        out_shape=jax.ShapeDtypeStruct((B, 1), jnp.float32),
    )(z, stats, gb, w2row, b2)


def kernel(x, tables, W1, gamma, beta, W2, b2):
    idxp = jnp.pad(x[:, :F].astype(jnp.int32), ((0, 0), (0, 128 - F)))
    emb = _sc_gather(tables, idxp)

    xdp = jnp.pad(x[:, F:], ((0, 0), (0, 16 - DENSE)))
    w1e = W1[:EMB_W]
    w1dp = jnp.pad(W1[EMB_W:], ((0, 16 - DENSE), (0, 0)))
    z, stats = _mlp1(emb, xdp, w1e, w1dp)

    gb = jnp.stack([gamma, beta], axis=0)
    w2row = W2.reshape(1, 128)
    b2m = b2.reshape(1, 1)
    return _mlp2(z, stats, gb, w2row, b2m)


# speedup vs baseline: 1.0355x; 1.0355x over previous
import functools

import jax
import jax.numpy as jnp
from jax import lax
from jax.experimental import pallas as pl
from jax.experimental.pallas import tpu as pltpu
from jax.experimental.pallas import tpu_sc as plsc

B = 16384
F = 26
DENSE = 13
V = 100000
H = 32
EMB_W = F * H
LOOKUPS = B * F

NC = 2
NS = 16
NW = NC * NS
CHUNK = 128
CPW = B // CHUNK // NW

BM = 1024
NB = B // BM


ROWS_W = B // NW
NL = 16


def _sc_gather(tables, idxp):
    mesh = plsc.VectorSubcoreMesh(core_axis_name="c", subcore_axis_name="s")

    @functools.partial(
        pl.kernel,
        mesh=mesh,
        compiler_params=pltpu.CompilerParams(
            use_tc_tiling_on_sc=False, needs_layout_passes=False),
        out_type=jax.ShapeDtypeStruct((B, EMB_W), jnp.float32),
        scratch_types=[
            pltpu.VMEM((ROWS_W, 128), jnp.int32),
            pltpu.VMEM((F, CPW, CHUNK), jnp.int32),
            pltpu.VMEM((CHUNK, H), jnp.float32),
            pltpu.VMEM((CHUNK, H), jnp.float32),
            pltpu.SemaphoreType.DMA,
            pltpu.SemaphoreType.DMA,
        ],
    )
    def k(tab_hbm, x_hbm, out_hbm, xv, idx_v, rows_a, rows_b, sem_a, sem_b):
        wid = lax.axis_index("s") * NC + lax.axis_index("c")
        b0 = wid * ROWS_W
        pltpu.sync_copy(x_hbm.at[pl.ds(b0, ROWS_W)], xv)

        lane = lax.iota(jnp.int32, NL)

        def build(t, carry):
            f = t // (ROWS_W // NL)
            g = t % (ROWS_W // NL)
            ids = plsc.load_gather(
                xv, [g * NL + lane, jnp.full((NL,), f, jnp.int32)])
            idx_v[f, g // (CHUNK // NL),
                  pl.ds((g % (CHUNK // NL)) * NL, NL)] = ids
            return carry

        lax.fori_loop(0, F * (ROWS_W // NL), build, 0)

        NT = F * CPW

        def start(t, rows, sem):
            f = t // CPW
            j = t % CPW
            pltpu.async_copy(tab_hbm.at[f].at[idx_v.at[f, j]], rows, sem)

        def drain(rows, sem):
            pltpu.make_async_copy(
                tab_hbm.at[0].at[idx_v.at[0, 0]], rows, sem).wait()

        def store(t, rows):
            f = t // CPW
            j = t % CPW
            pltpu.sync_copy(
                rows,
                out_hbm.at[pl.ds(b0 + j * CHUNK, CHUNK), pl.ds(f * H, H)])

        start(0, rows_a, sem_a)

        def pair(p, carry):
            t0 = 2 * p
            start(t0 + 1, rows_b, sem_b)
            drain(rows_a, sem_a)
            store(t0, rows_a)

            @pl.when(t0 + 2 < NT)
            def _():
                start(t0 + 2, rows_a, sem_a)

            drain(rows_b, sem_b)
            store(t0 + 1, rows_b)
            return carry

        lax.fori_loop(0, NT // 2, pair, 0)

    return k(tables, idxp)


def _mlp1_body(emb_ref, xd_ref, w1e_ref, w1d_ref, z_ref, stats_ref):
    j = pl.program_id(0)
    z = jnp.dot(emb_ref[...], w1e_ref[...], preferred_element_type=jnp.float32)
    z = z + jnp.dot(xd_ref[...], w1d_ref[...], preferred_element_type=jnp.float32)
    z_ref[...] = z
    s1 = jnp.sum(z, axis=0, keepdims=True)
    s2 = jnp.sum(z * z, axis=0, keepdims=True)

    @pl.when(j == 0)
    def _():
        stats_ref[...] = jnp.zeros_like(stats_ref)

    stats_ref[...] += jnp.concatenate(
        [s1, s2, jnp.zeros((6, 128), jnp.float32)], axis=0)


def _mlp1(emb, xdp, w1e, w1dp):
    return pl.pallas_call(
        _mlp1_body,
        grid=(NB,),
        in_specs=[
            pl.BlockSpec((BM, EMB_W), lambda j: (j, 0)),
            pl.BlockSpec((BM, 16), lambda j: (j, 0)),
            pl.BlockSpec((EMB_W, 128), lambda j: (0, 0)),
            pl.BlockSpec((16, 128), lambda j: (0, 0)),
        ],
        out_specs=[
            pl.BlockSpec((BM, 128), lambda j: (j, 0)),
            pl.BlockSpec((8, 128), lambda j: (0, 0)),
        ],
        out_shape=[
            jax.ShapeDtypeStruct((B, 128), jnp.float32),
            jax.ShapeDtypeStruct((8, 128), jnp.float32),
        ],
    )(emb, xdp, w1e, w1dp)


def _mlp2_body(z_ref, stats_ref, gb_ref, w2_ref, b2_ref, out_ref):
    stats = stats_ref[...]
    mean = stats[0:1] / B
    var = stats[1:2] / B - mean * mean
    scale = gb_ref[0:1] * lax.rsqrt(var + 1e-5)
    shift = gb_ref[1:2] - mean * scale
    a = jnp.maximum(z_ref[...] * scale + shift, 0.0)
    o = jnp.sum(a * w2_ref[...], axis=1, keepdims=True) + b2_ref[...]
    out_ref[...] = jax.nn.sigmoid(o)


def _mlp2(z, stats, gb, w2row, b2):
    return pl.pallas_call(
        _mlp2_body,
        grid=(NB,),
        in_specs=[
            pl.BlockSpec((BM, 128), lambda j: (j, 0)),
            pl.BlockSpec((8, 128), lambda j: (0, 0)),
            pl.BlockSpec((2, 128), lambda j: (0, 0)),
            pl.BlockSpec((1, 128), lambda j: (0, 0)),
            pl.BlockSpec((1, 1), lambda j: (0, 0)),
        ],
        out_specs=pl.BlockSpec((BM, 1), lambda j: (j, 0)),
        out_shape=jax.ShapeDtypeStruct((B, 1), jnp.float32),
    )(z, stats, gb, w2row, b2)


def kernel(x, tables, W1, gamma, beta, W2, b2):
    idxp = jnp.pad(x[:, :F].astype(jnp.int32), ((0, 0), (0, 128 - F)))
    emb = _sc_gather(tables, idxp)

    xdp = jnp.pad(x[:, F:], ((0, 0), (0, 16 - DENSE)))
    w1e = W1[:EMB_W]
    w1dp = jnp.pad(W1[EMB_W:], ((0, 16 - DENSE), (0, 0)))
    z, stats = _mlp1(emb, xdp, w1e, w1dp)

    gb = jnp.stack([gamma, beta], axis=0)
    w2row = W2.reshape(1, 128)
    b2m = b2.reshape(1, 1)
    return _mlp2(z, stats, gb, w2row, b2m)


# speedup vs baseline: 1.0558x; 1.0195x over previous
import functools

import jax
import jax.numpy as jnp
from jax import lax
from jax.experimental import pallas as pl
from jax.experimental.pallas import tpu as pltpu
from jax.experimental.pallas import tpu_sc as plsc

B = 16384
F = 26
DENSE = 13
V = 100000
H = 32
EMB_W = F * H
LOOKUPS = B * F

NC = 2
NS = 16
NW = NC * NS
CHUNK = 128
CPW = B // CHUNK // NW

BM = 1024
NB = B // BM


ROWS_W = B // NW
NL = 16


def _sc_gather(tables, idxp):
    mesh = plsc.VectorSubcoreMesh(core_axis_name="c", subcore_axis_name="s")

    @functools.partial(
        pl.kernel,
        mesh=mesh,
        compiler_params=pltpu.CompilerParams(
            use_tc_tiling_on_sc=False, needs_layout_passes=False),
        out_type=jax.ShapeDtypeStruct((B, EMB_W), jnp.float32),
        scratch_types=[
            pltpu.VMEM((ROWS_W, 128), jnp.int32),
            pltpu.VMEM((F, CPW, CHUNK), jnp.int32),
            pltpu.VMEM((CHUNK, H), jnp.float32),
            pltpu.VMEM((CHUNK, H), jnp.float32),
            pltpu.VMEM((CHUNK, H), jnp.float32),
            pltpu.VMEM((CHUNK, H), jnp.float32),
            pltpu.SemaphoreType.DMA,
            pltpu.SemaphoreType.DMA,
            pltpu.SemaphoreType.DMA,
            pltpu.SemaphoreType.DMA,
        ],
    )
    def k(tab_hbm, x_hbm, out_hbm, xv, idx_v,
          rows_a, rows_b, rows_c, rows_d, sem_a, sem_b, sem_c, sem_d):
        wid = lax.axis_index("s") * NC + lax.axis_index("c")
        b0 = wid * ROWS_W
        pltpu.sync_copy(x_hbm.at[pl.ds(b0, ROWS_W)], xv)

        lane = lax.iota(jnp.int32, NL)

        def build(t, carry):
            f = t // (ROWS_W // NL)
            g = t % (ROWS_W // NL)
            ids = plsc.load_gather(
                xv, [g * NL + lane, jnp.full((NL,), f, jnp.int32)])
            idx_v[f, g // (CHUNK // NL),
                  pl.ds((g % (CHUNK // NL)) * NL, NL)] = ids
            return carry

        lax.fori_loop(0, F * (ROWS_W // NL), build, 0)

        NT = F * CPW

        def start(t, rows, sem):
            f = t // CPW
            j = t % CPW
            pltpu.async_copy(tab_hbm.at[f].at[idx_v.at[f, j]], rows, sem)

        def drain(rows, sem):
            pltpu.make_async_copy(
                tab_hbm.at[0].at[idx_v.at[0, 0]], rows, sem).wait()

        def store(t, rows):
            f = t // CPW
            j = t % CPW
            pltpu.sync_copy(
                rows,
                out_hbm.at[pl.ds(b0 + j * CHUNK, CHUNK), pl.ds(f * H, H)])

        bufs = ((rows_a, sem_a), (rows_b, sem_b),
                (rows_c, sem_c), (rows_d, sem_d))
        for i in range(4):
            start(i, *bufs[i])

        def quad(p, carry):
            base = 4 * p
            for i in range(4):
                t = base + i
                rows, sem = bufs[i]
                drain(rows, sem)
                store(t, rows)

                @pl.when(t + 4 < NT)
                def _(t=t, rows=rows, sem=sem):
                    start(t + 4, rows, sem)

            return carry

        lax.fori_loop(0, NT // 4, quad, 0)

    return k(tables, idxp)


def _mlp1_body(emb_ref, xd_ref, w1e_ref, w1d_ref, z_ref, stats_ref):
    j = pl.program_id(0)
    z = jnp.dot(emb_ref[...], w1e_ref[...], preferred_element_type=jnp.float32)
    z = z + jnp.dot(xd_ref[...], w1d_ref[...], preferred_element_type=jnp.float32)
    z_ref[...] = z
    s1 = jnp.sum(z, axis=0, keepdims=True)
    s2 = jnp.sum(z * z, axis=0, keepdims=True)

    @pl.when(j == 0)
    def _():
        stats_ref[...] = jnp.zeros_like(stats_ref)

    stats_ref[...] += jnp.concatenate(
        [s1, s2, jnp.zeros((6, 128), jnp.float32)], axis=0)


def _mlp1(emb, xdp, w1e, w1dp):
    return pl.pallas_call(
        _mlp1_body,
        grid=(NB,),
        in_specs=[
            pl.BlockSpec((BM, EMB_W), lambda j: (j, 0)),
            pl.BlockSpec((BM, 16), lambda j: (j, 0)),
            pl.BlockSpec((EMB_W, 128), lambda j: (0, 0)),
            pl.BlockSpec((16, 128), lambda j: (0, 0)),
        ],
        out_specs=[
            pl.BlockSpec((BM, 128), lambda j: (j, 0)),
            pl.BlockSpec((8, 128), lambda j: (0, 0)),
        ],
        out_shape=[
            jax.ShapeDtypeStruct((B, 128), jnp.float32),
            jax.ShapeDtypeStruct((8, 128), jnp.float32),
        ],
    )(emb, xdp, w1e, w1dp)


def _mlp2_body(z_ref, stats_ref, gb_ref, w2_ref, b2_ref, out_ref):
    stats = stats_ref[...]
    mean = stats[0:1] / B
    var = stats[1:2] / B - mean * mean
    scale = gb_ref[0:1] * lax.rsqrt(var + 1e-5)
    shift = gb_ref[1:2] - mean * scale
    a = jnp.maximum(z_ref[...] * scale + shift, 0.0)
    o = jnp.sum(a * w2_ref[...], axis=1, keepdims=True) + b2_ref[...]
    out_ref[...] = jax.nn.sigmoid(o)


def _mlp2(z, stats, gb, w2row, b2):
    return pl.pallas_call(
        _mlp2_body,
        grid=(NB,),
        in_specs=[
            pl.BlockSpec((BM, 128), lambda j: (j, 0)),
            pl.BlockSpec((8, 128), lambda j: (0, 0)),
            pl.BlockSpec((2, 128), lambda j: (0, 0)),
            pl.BlockSpec((1, 128), lambda j: (0, 0)),
            pl.BlockSpec((1, 1), lambda j: (0, 0)),
        ],
        out_specs=pl.BlockSpec((BM, 1), lambda j: (j, 0)),
        out_shape=jax.ShapeDtypeStruct((B, 1), jnp.float32),
    )(z, stats, gb, w2row, b2)


def kernel(x, tables, W1, gamma, beta, W2, b2):
    idxp = jnp.pad(x[:, :F].astype(jnp.int32), ((0, 0), (0, 128 - F)))
    emb = _sc_gather(tables, idxp)

    xdp = jnp.pad(x[:, F:], ((0, 0), (0, 16 - DENSE)))
    w1e = W1[:EMB_W]
    w1dp = jnp.pad(W1[EMB_W:], ((0, 16 - DENSE), (0, 0)))
    z, stats = _mlp1(emb, xdp, w1e, w1dp)

    gb = jnp.stack([gamma, beta], axis=0)
    w2row = W2.reshape(1, 128)
    b2m = b2.reshape(1, 1)
    return _mlp2(z, stats, gb, w2row, b2m)
